# baseline (speedup 1.0000x reference)
import functools

import jax
import jax.numpy as jnp
from jax import lax
from jax.experimental import pallas as pl
from jax.experimental.pallas import tpu as pltpu
from jax.experimental.pallas import tpu_sc as plsc

N = 10000
D = 256
H = 4
DH = 64
E = 160000
NC = 2
NS = 16
NPAD = 10240
EPAD = 163840
EPT = EPAD // NS
CB = 64
NPAIR = EPT // (2 * CB)
RPT = NPAD // NS
ORT = N // NS
BLK = 1000
GRID = N // BLK


def _s1_body(xa, xo, wa, wo, ba, bo, psa, pda, pso, pdo,
             ha0, ha1, ho0, ho1, tsa, tda, tso, tdo):
    ha = jnp.dot(xa[...], wa[...], preferred_element_type=jnp.float32) + ba[...]
    ho = jnp.dot(xo[...], wo[...], preferred_element_type=jnp.float32) + bo[...]
    ha0[...] = ha[:, :128]
    ha1[...] = ha[:, 128:]
    ho0[...] = ho[:, :128]
    ho1[...] = ho[:, 128:]
    tsa[...] = jnp.dot(ha, psa[...], preferred_element_type=jnp.float32)
    tda[...] = jnp.dot(ha, pda[...], preferred_element_type=jnp.float32)
    tso[...] = jnp.dot(ho, pso[...], preferred_element_type=jnp.float32)
    tdo[...] = jnp.dot(ha, pdo[...], preferred_element_type=jnp.float32)


def _stage1(xa, xo, wa, wo, ba, bo, psa, pda, pso, pdo):
    row = pl.BlockSpec((BLK, D), lambda i: (i, 0))
    full = lambda s: pl.BlockSpec(s, lambda i: (0, 0))
    outs = [jax.ShapeDtypeStruct((N, 128), jnp.float32)] * 4 + \
           [jax.ShapeDtypeStruct((N, 16), jnp.float32)] * 4
    return pl.pallas_call(
        _s1_body,
        grid=(GRID,),
        in_specs=[row, row, full((D, D)), full((D, D)), full((1, D)),
                  full((1, D)), full((D, 16)), full((D, 16)), full((D, 16)),
                  full((D, 16))],
        out_specs=[pl.BlockSpec((BLK, 128), lambda i: (i, 0))] * 4 +
                  [pl.BlockSpec((BLK, 16), lambda i: (i, 0))] * 4,
        out_shape=outs,
    )(xa, xo, wa, wo, ba, bo, psa, pda, pso, pdo)


def _run_type(c, s, ei, tsrc, tdst, href, exref, recref, aggref,
              den, acc, sb0, db0, sb1, db1, ar0, br0, ar1, br1,
              ex0b, ex1b, rc0, rc1, hr0, hr1, semA, semB):
    ebase = s * EPT
    row0 = s * RPT

    @plsc.parallel_loop(0, CB, unroll=8)
    def zi(i):
        for u in range(8):
            hr0[i, pl.ds(u * 16, 16)] = jnp.zeros((16,), jnp.float32)
        ar0[i, :] = jnp.zeros((16,), jnp.float32)
    for k in range(RPT // CB):
        pltpu.sync_copy(ar0, den.at[pl.ds(row0 + k * CB, CB), :])
        pltpu.sync_copy(hr0, acc.at[pl.ds(row0 + k * CB, CB), :])
    plsc.subcore_barrier()

    def p1(p, carry):
        e0 = ebase + p * 2 * CB
        e1 = e0 + CB
        pltpu.sync_copy(ei.at[0, pl.ds(e0, CB)], sb0)
        pltpu.sync_copy(ei.at[1, pl.ds(e0, CB)], db0)
        pltpu.sync_copy(ei.at[0, pl.ds(e1, CB)], sb1)
        pltpu.sync_copy(ei.at[1, pl.ds(e1, CB)], db1)
        ca0 = pltpu.async_copy(tsrc.at[sb0], ar0, semA)
        cb0 = pltpu.async_copy(tdst.at[db0], br0, semA)
        ca1 = pltpu.async_copy(tsrc.at[sb1], ar1, semB)
        cb1 = pltpu.async_copy(tdst.at[db1], br1, semB)
        ca0.wait()
        cb0.wait()

        @plsc.parallel_loop(0, CB, unroll=8)
        def cmp0(i):
            a = ar0[i, :] + br0[i, :]
            a = jnp.maximum(a, 0.2 * a)
            ex0b[i, :] = jnp.exp(a)
        pltpu.sync_copy(ex0b, exref.at[pl.ds(e0, CB), :])
        pltpu.sync_copy(ex0b, den.at[db0], add=True)
        ca1.wait()
        cb1.wait()

        @plsc.parallel_loop(0, CB, unroll=8)
        def cmp1(i):
            a = ar1[i, :] + br1[i, :]
            a = jnp.maximum(a, 0.2 * a)
            ex1b[i, :] = jnp.exp(a)
        pltpu.sync_copy(ex1b, exref.at[pl.ds(e1, CB), :])
        pltpu.sync_copy(ex1b, den.at[db1], add=True)
        return carry
    lax.fori_loop(0, NPAIR, p1, 0)
    plsc.subcore_barrier()

    for k in range(RPT // CB):
        r0 = row0 + k * CB
        pltpu.sync_copy(den.at[pl.ds(r0, CB), :], ar0)

        @plsc.parallel_loop(0, CB, unroll=8)
        def rcp(i):
            ar0[i, :] = 1.0 / (ar0[i, :] + 1e-16)
        pltpu.sync_copy(ar0, recref.at[pl.ds(r0, CB), :])
    plsc.subcore_barrier()

    h0 = 2 * c
    i0 = jnp.broadcast_to(h0, (16, 1)).astype(jnp.int32)
    i1 = jnp.broadcast_to(h0 + 1, (16, 1)).astype(jnp.int32)
    _dnums = lax.GatherDimensionNumbers(
        offset_dims=(), collapsed_slice_dims=(0,), start_index_map=(0,))
    _splat = functools.partial(
        lax.gather, dimension_numbers=_dnums, slice_sizes=(1,),
        mode=lax.GatherScatterMode.PROMISE_IN_BOUNDS)

    def p3(p, carry):
        e0 = ebase + p * 2 * CB
        e1 = e0 + CB
        pltpu.sync_copy(ei.at[0, pl.ds(e0, CB)], sb0)
        pltpu.sync_copy(ei.at[1, pl.ds(e0, CB)], db0)
        pltpu.sync_copy(ei.at[0, pl.ds(e1, CB)], sb1)
        pltpu.sync_copy(ei.at[1, pl.ds(e1, CB)], db1)
        ch0 = pltpu.async_copy(href.at[sb0], hr0, semA)
        cr0 = pltpu.async_copy(recref.at[db0], rc0, semA)
        ce0 = pltpu.async_copy(exref.at[pl.ds(e0, CB), :], ex0b, semA)
        ch1 = pltpu.async_copy(href.at[sb1], hr1, semB)
        cr1 = pltpu.async_copy(recref.at[db1], rc1, semB)
        ce1 = pltpu.async_copy(exref.at[pl.ds(e1, CB), :], ex1b, semB)
        ch0.wait()
        cr0.wait()
        ce0.wait()

        @plsc.parallel_loop(0, CB, unroll=4)
        def scale0(k):
            row = ex0b[k, :] * rc0[k, :]
            s0 = _splat(row, i0)
            s1 = _splat(row, i1)
            for u in range(4):
                hr0[k, pl.ds(u * 16, 16)] = hr0[k, pl.ds(u * 16, 16)] * s0
            for u in range(4, 8):
                hr0[k, pl.ds(u * 16, 16)] = hr0[k, pl.ds(u * 16, 16)] * s1
        pltpu.sync_copy(hr0, acc.at[db0], add=True)
        ch1.wait()
        cr1.wait()
        ce1.wait()

        @plsc.parallel_loop(0, CB, unroll=4)
        def scale1(k):
            row = ex1b[k, :] * rc1[k, :]
            s0 = _splat(row, i0)
            s1 = _splat(row, i1)
            for u in range(4):
                hr1[k, pl.ds(u * 16, 16)] = hr1[k, pl.ds(u * 16, 16)] * s0
            for u in range(4, 8):
                hr1[k, pl.ds(u * 16, 16)] = hr1[k, pl.ds(u * 16, 16)] * s1
        pltpu.sync_copy(hr1, acc.at[db1], add=True)
        return carry
    lax.fori_loop(0, NPAIR, p3, 0)
    plsc.subcore_barrier()

    for k in range(RPT // CB):
        r0 = row0 + k * CB
        pltpu.sync_copy(acc.at[pl.ds(r0, CB), :], hr0)
        pltpu.sync_copy(hr0, aggref.at[pl.ds(r0, CB), :])
    plsc.subcore_barrier()


def _sc_body(ei_ab, ei_ob, tsa, tda, tso, tdo, ha0, ha1, ho0, ho1,
             gab0, gab1, gob0, gob1, ex0, ex1, rec0, rec1,
             den, acc, sb0, db0, sb1, db1, ar0, br0, ar1, br1,
             ex0b, ex1b, rc0, rc1, hr0, hr1, semA, semB):
    c = lax.axis_index("c")
    s = lax.axis_index("s")

    common = (den, acc, sb0, db0, sb1, db1, ar0, br0, ar1, br1,
              ex0b, ex1b, rc0, rc1, hr0, hr1, semA, semB)

    @pl.when(c == 0)
    def _():
        _run_type(c, s, ei_ab, tsa, tda, ha0, ex0, rec0, gab0, *common)
        _run_type(c, s, ei_ob, tso, tdo, ho0, ex0, rec0, gob0, *common)

    @pl.when(c == 1)
    def _():
        _run_type(c, s, ei_ab, tsa, tda, ha1, ex1, rec1, gab1, *common)
        _run_type(c, s, ei_ob, tso, tdo, ho1, ex1, rec1, gob1, *common)


def _stage2(ei_ab, ei_ob, tsa, tda, tso, tdo, ha0, ha1, ho0, ho1):
    mesh = plsc.VectorSubcoreMesh(core_axis_name="c", subcore_axis_name="s",
                                  num_cores=NC, num_subcores=NS)
    f32 = jnp.float32
    out_type = (
        jax.ShapeDtypeStruct((NPAD, 128), f32),
        jax.ShapeDtypeStruct((NPAD, 128), f32),
        jax.ShapeDtypeStruct((NPAD, 128), f32),
        jax.ShapeDtypeStruct((NPAD, 128), f32),
        jax.ShapeDtypeStruct((EPAD, 16), f32),
        jax.ShapeDtypeStruct((EPAD, 16), f32),
        jax.ShapeDtypeStruct((NPAD, 16), f32),
        jax.ShapeDtypeStruct((NPAD, 16), f32),
    )
    scratch = [
        pltpu.VMEM_SHARED((NPAD, 16), f32),
        pltpu.VMEM_SHARED((NPAD, 128), f32),
        pltpu.VMEM((CB,), jnp.int32),
        pltpu.VMEM((CB,), jnp.int32),
        pltpu.VMEM((CB,), jnp.int32),
        pltpu.VMEM((CB,), jnp.int32),
        pltpu.VMEM((CB, 16), f32),
        pltpu.VMEM((CB, 16), f32),
        pltpu.VMEM((CB, 16), f32),
        pltpu.VMEM((CB, 16), f32),
        pltpu.VMEM((CB, 16), f32),
        pltpu.VMEM((CB, 16), f32),
        pltpu.VMEM((CB, 16), f32),
        pltpu.VMEM((CB, 16), f32),
        pltpu.VMEM((CB, 128), f32),
        pltpu.VMEM((CB, 128), f32),
        pltpu.SemaphoreType.DMA,
        pltpu.SemaphoreType.DMA,
    ]
    fn = pl.kernel(_sc_body, out_type=out_type, mesh=mesh,
                   scratch_types=scratch,
                   compiler_params=pltpu.CompilerParams(
                       use_tc_tiling_on_sc=False))
    return fn(ei_ab, ei_ob, tsa, tda, tso, tdo, ha0, ha1, ho0, ho1)


def _s3a_body(ab0, ab1, ob0, ob1, wk, bk, ksum):
    i = pl.program_id(0)
    rab = jnp.maximum(jnp.concatenate([ab0[...], ab1[...]], axis=1), 0.0)
    rob = jnp.maximum(jnp.concatenate([ob0[...], ob1[...]], axis=1), 0.0)
    tab = jnp.tanh(jnp.dot(rab, wk[...], preferred_element_type=jnp.float32) + bk[...])
    tob = jnp.tanh(jnp.dot(rob, wk[...], preferred_element_type=jnp.float32) + bk[...])
    blk = jnp.concatenate([jnp.sum(tab, axis=0, keepdims=True),
                           jnp.sum(tob, axis=0, keepdims=True)], axis=0)

    @pl.when(i == 0)
    def _():
        ksum[...] = blk

    @pl.when(i > 0)
    def _():
        ksum[...] = ksum[...] + blk


def _s3b_body(ksum, qv, ab0, ab1, ob0, ob1, out):
    km = ksum[...] * (1.0 / N)
    s = jnp.sum(km * qv[...], axis=1, keepdims=True)
    m = jnp.max(s)
    e = jnp.exp(s - m)
    at = e / jnp.sum(e)
    a0 = at[0, 0]
    a1 = at[1, 0]
    rab = jnp.maximum(jnp.concatenate([ab0[...], ab1[...]], axis=1), 0.0)
    rob = jnp.maximum(jnp.concatenate([ob0[...], ob1[...]], axis=1), 0.0)
    out[...] = a0 * rab + a1 * rob


def _stage3(gab0, gab1, gob0, gob1, wk, bk, qv):
    rowh = pl.BlockSpec((BLK, 128), lambda i: (i, 0))
    full = lambda s: pl.BlockSpec(s, lambda i: (0, 0))
    ksum = pl.pallas_call(
        _s3a_body,
        grid=(GRID,),
        in_specs=[rowh, rowh, rowh, rowh, full((D, D)), full((1, D))],
        out_specs=pl.BlockSpec((2, D), lambda i: (0, 0)),
        out_shape=jax.ShapeDtypeStruct((2, D), jnp.float32),
    )(gab0, gab1, gob0, gob1, wk, bk)
    return pl.pallas_call(
        _s3b_body,
        grid=(GRID,),
        in_specs=[full((2, D)), full((1, D)), rowh, rowh, rowh, rowh],
        out_specs=pl.BlockSpec((BLK, D), lambda i: (i, 0)),
        out_shape=jax.ShapeDtypeStruct((N, D), jnp.float32),
    )(ksum, qv, gab0, gab1, gob0, gob1)


def _proj_mat(a):
    P = jnp.zeros((D, 16), jnp.float32)
    return P.at[jnp.arange(D), jnp.repeat(jnp.arange(H), DH)].set(a.reshape(-1))


def _pad_edges(ei):
    npad = EPAD - E
    src = (jnp.arange(npad, dtype=jnp.int32) * 7) % N
    dst = N + (jnp.arange(npad, dtype=jnp.int32) % 128)
    return jnp.concatenate([ei, jnp.stack([src, dst])], axis=1)


def kernel(x_abc_stock, x_other, edge_index_abc_to_abc, edge_index_other_to_abc,
           W_abc, b_abc, W_other, b_other, a_src_ab, a_dst_ab, a_src_ob,
           a_dst_ob, Wk, bk, q, emb_weight):
    ha0, ha1, ho0, ho1, tsa, tda, tso, tdo = _stage1(
        x_abc_stock, x_other, W_abc, W_other,
        b_abc.reshape(1, D), b_other.reshape(1, D),
        _proj_mat(a_src_ab), _proj_mat(a_dst_ab),
        _proj_mat(a_src_ob), _proj_mat(a_dst_ob))
    ei_ab = _pad_edges(edge_index_abc_to_abc)
    ei_ob = _pad_edges(edge_index_other_to_abc)
    gab0, gab1, gob0, gob1 = [g[:N] for g in _stage2(
        ei_ab, ei_ob, tsa, tda, tso, tdo, ha0, ha1, ho0, ho1)[:4]]
    out = _stage3(gab0, gab1, gob0, gob1, Wk, bk.reshape(1, D),
                  q.reshape(1, D))
    return out, emb_weight


# speedup vs baseline: 1.5794x; 1.5794x over previous
import functools

import jax
import jax.numpy as jnp
from jax import lax
from jax.experimental import pallas as pl
from jax.experimental.pallas import tpu as pltpu
from jax.experimental.pallas import tpu_sc as plsc

N = 10000
D = 256
H = 4
DH = 64
E = 160000
NC = 2
NS = 16
NPAD = 10240
EPAD = 163840
EPT = EPAD // NS
ECH = 128
NCHUNK = EPT // ECH
GRP = 8
NGRP = NCHUNK // GRP
RPT = NPAD // NS
ORT = N // NS
BLK = 1000
GRID = N // BLK


def _s1_body(xa, xo, wa, wo, ba, bo, psa, pda, pso, pdo,
             ha0, ha1, ho0, ho1, tsa, tda, tso, tdo):
    ha = jnp.dot(xa[...], wa[...], preferred_element_type=jnp.float32) + ba[...]
    ho = jnp.dot(xo[...], wo[...], preferred_element_type=jnp.float32) + bo[...]
    ha0[...] = ha[:, :128]
    ha1[...] = ha[:, 128:]
    ho0[...] = ho[:, :128]
    ho1[...] = ho[:, 128:]
    tsa[...] = jnp.dot(ha, psa[...], preferred_element_type=jnp.float32)
    tda[...] = jnp.dot(ha, pda[...], preferred_element_type=jnp.float32)
    tso[...] = jnp.dot(ho, pso[...], preferred_element_type=jnp.float32)
    tdo[...] = jnp.dot(ha, pdo[...], preferred_element_type=jnp.float32)


def _stage1(xa, xo, wa, wo, ba, bo, psa, pda, pso, pdo):
    row = pl.BlockSpec((BLK, D), lambda i: (i, 0))
    full = lambda s: pl.BlockSpec(s, lambda i: (0, 0))
    outs = [jax.ShapeDtypeStruct((N, 128), jnp.float32)] * 4 + \
           [jax.ShapeDtypeStruct((N, 16), jnp.float32)] * 4
    return pl.pallas_call(
        _s1_body,
        grid=(GRID,),
        in_specs=[row, row, full((D, D)), full((D, D)), full((1, D)),
                  full((1, D)), full((D, 16)), full((D, 16)), full((D, 16)),
                  full((D, 16))],
        out_specs=[pl.BlockSpec((BLK, 128), lambda i: (i, 0))] * 4 +
                  [pl.BlockSpec((BLK, 16), lambda i: (i, 0))] * 4,
        out_shape=outs,
    )(xa, xo, wa, wo, ba, bo, psa, pda, pso, pdo)


def _run_type(c, s, ei, tsrc, tdst, href, exref, recref, aggref,
              den, acc, sbuf, dbuf, arows, brows, exch, rech, hrows,
              sem, sem2):
    ebase = s * EPT
    row0 = s * RPT

    @plsc.parallel_loop(0, 128, unroll=8)
    def zi(i):
        for u in range(8):
            hrows[i, pl.ds(u * 16, 16)] = jnp.zeros((16,), jnp.float32)
        arows[i, :] = jnp.zeros((16,), jnp.float32)
    for k in range(RPT // 128):
        pltpu.sync_copy(arows, den.at[pl.ds(row0 + k * 128, 128), :])
        pltpu.sync_copy(hrows, acc.at[pl.ds(row0 + k * 128, 128), :])
    plsc.subcore_barrier()

    crow0 = (s * EPT) // ECH

    def p1(g, carry):
        gr = crow0 + g * GRP
        pltpu.sync_copy(ei.at[0, pl.ds(gr, GRP), :], sbuf)
        pltpu.sync_copy(ei.at[1, pl.ds(gr, GRP), :], dbuf)

        def p1c(j, cc):
            eb = (gr + j) * ECH
            ca = pltpu.async_copy(tsrc.at[sbuf.at[j]], arows, sem)
            cb = pltpu.async_copy(tdst.at[dbuf.at[j]], brows, sem2)
            ca.wait()
            cb.wait()

            @plsc.parallel_loop(0, ECH, unroll=8)
            def cmp(i):
                a = arows[i, :] + brows[i, :]
                a = jnp.maximum(a, 0.2 * a)
                exch[i, :] = jnp.exp(a)
            pltpu.sync_copy(exch, exref.at[pl.ds(eb, ECH), :])
            pltpu.sync_copy(exch, den.at[dbuf.at[j]], add=True)
            return cc
        lax.fori_loop(0, GRP, p1c, 0)
        return carry
    lax.fori_loop(0, NGRP, p1, 0)
    plsc.subcore_barrier()

    for k in range(RPT // 128):
        r0 = row0 + k * 128
        pltpu.sync_copy(den.at[pl.ds(r0, 128), :], arows)

        @plsc.parallel_loop(0, 128, unroll=8)
        def rcp(i):
            arows[i, :] = 1.0 / (arows[i, :] + 1e-16)
        pltpu.sync_copy(arows, recref.at[pl.ds(r0, 128), :])
    plsc.subcore_barrier()

    h0 = 2 * c
    i0 = jnp.broadcast_to(h0, (16, 1)).astype(jnp.int32)
    i1 = jnp.broadcast_to(h0 + 1, (16, 1)).astype(jnp.int32)
    _dnums = lax.GatherDimensionNumbers(
        offset_dims=(), collapsed_slice_dims=(0,), start_index_map=(0,))
    _splat = functools.partial(
        lax.gather, dimension_numbers=_dnums, slice_sizes=(1,),
        mode=lax.GatherScatterMode.PROMISE_IN_BOUNDS)

    def p3(g, carry):
        gr = crow0 + g * GRP
        pltpu.sync_copy(ei.at[0, pl.ds(gr, GRP), :], sbuf)
        pltpu.sync_copy(ei.at[1, pl.ds(gr, GRP), :], dbuf)

        def p3c(j, cc):
            eb = (gr + j) * ECH
            cp = pltpu.async_copy(href.at[sbuf.at[j]], hrows, sem)
            cr = pltpu.async_copy(recref.at[dbuf.at[j]], rech, sem2)
            pltpu.sync_copy(exref.at[pl.ds(eb, ECH), :], exch)
            cr.wait()
            cp.wait()

            @plsc.parallel_loop(0, ECH, unroll=4)
            def scale(k):
                row = exch[k, :] * rech[k, :]
                s0 = _splat(row, i0)
                s1 = _splat(row, i1)
                for u in range(4):
                    hrows[k, pl.ds(u * 16, 16)] = hrows[k, pl.ds(u * 16, 16)] * s0
                for u in range(4, 8):
                    hrows[k, pl.ds(u * 16, 16)] = hrows[k, pl.ds(u * 16, 16)] * s1
            pltpu.sync_copy(hrows, acc.at[dbuf.at[j]], add=True)
            return cc
        lax.fori_loop(0, GRP, p3c, 0)
        return carry
    lax.fori_loop(0, NGRP, p3, 0)
    plsc.subcore_barrier()

    for k in range(RPT // 128):
        r0 = row0 + k * 128
        pltpu.sync_copy(acc.at[pl.ds(r0, 128), :], hrows)
        pltpu.sync_copy(hrows, aggref.at[pl.ds(r0, 128), :])
    plsc.subcore_barrier()


def _sc_body(ei_ab, ei_ob, tsa, tda, tso, tdo, ha0, ha1, ho0, ho1,
             gab0, gab1, gob0, gob1, ex0, ex1, rec0, rec1,
             den, acc, sbuf, dbuf, arows, brows, exch, rech, hrows,
             sem, sem2):
    c = lax.axis_index("c")
    s = lax.axis_index("s")

    common = (den, acc, sbuf, dbuf, arows, brows, exch, rech, hrows,
              sem, sem2)

    @pl.when(c == 0)
    def _():
        _run_type(c, s, ei_ab, tsa, tda, ha0, ex0, rec0, gab0, *common)
        _run_type(c, s, ei_ob, tso, tdo, ho0, ex0, rec0, gob0, *common)

    @pl.when(c == 1)
    def _():
        _run_type(c, s, ei_ab, tsa, tda, ha1, ex1, rec1, gab1, *common)
        _run_type(c, s, ei_ob, tso, tdo, ho1, ex1, rec1, gob1, *common)


def _stage2(ei_ab, ei_ob, tsa, tda, tso, tdo, ha0, ha1, ho0, ho1):
    mesh = plsc.VectorSubcoreMesh(core_axis_name="c", subcore_axis_name="s",
                                  num_cores=NC, num_subcores=NS)
    f32 = jnp.float32
    out_type = (
        jax.ShapeDtypeStruct((NPAD, 128), f32),
        jax.ShapeDtypeStruct((NPAD, 128), f32),
        jax.ShapeDtypeStruct((NPAD, 128), f32),
        jax.ShapeDtypeStruct((NPAD, 128), f32),
        jax.ShapeDtypeStruct((EPAD, 16), f32),
        jax.ShapeDtypeStruct((EPAD, 16), f32),
        jax.ShapeDtypeStruct((NPAD, 16), f32),
        jax.ShapeDtypeStruct((NPAD, 16), f32),
    )
    scratch = [
        pltpu.VMEM_SHARED((NPAD, 16), f32),
        pltpu.VMEM_SHARED((NPAD, 128), f32),
        pltpu.VMEM((GRP, ECH), jnp.int32),
        pltpu.VMEM((GRP, ECH), jnp.int32),
        pltpu.VMEM((ECH, 16), f32),
        pltpu.VMEM((ECH, 16), f32),
        pltpu.VMEM((ECH, 16), f32),
        pltpu.VMEM((ECH, 16), f32),
        pltpu.VMEM((ECH, 128), f32),
        pltpu.SemaphoreType.DMA,
        pltpu.SemaphoreType.DMA,
    ]
    fn = pl.kernel(_sc_body, out_type=out_type, mesh=mesh,
                   scratch_types=scratch,
                   compiler_params=pltpu.CompilerParams(
                       use_tc_tiling_on_sc=False))
    return fn(ei_ab, ei_ob, tsa, tda, tso, tdo, ha0, ha1, ho0, ho1)


def _s3a_body(ab0, ab1, ob0, ob1, wk, bk, ksum):
    i = pl.program_id(0)
    rab = jnp.maximum(jnp.concatenate([ab0[...], ab1[...]], axis=1), 0.0)
    rob = jnp.maximum(jnp.concatenate([ob0[...], ob1[...]], axis=1), 0.0)
    tab = jnp.tanh(jnp.dot(rab, wk[...], preferred_element_type=jnp.float32) + bk[...])
    tob = jnp.tanh(jnp.dot(rob, wk[...], preferred_element_type=jnp.float32) + bk[...])
    blk = jnp.concatenate([jnp.sum(tab, axis=0, keepdims=True),
                           jnp.sum(tob, axis=0, keepdims=True)], axis=0)

    @pl.when(i == 0)
    def _():
        ksum[...] = blk

    @pl.when(i > 0)
    def _():
        ksum[...] = ksum[...] + blk


def _s3b_body(ksum, qv, ab0, ab1, ob0, ob1, out):
    km = ksum[...] * (1.0 / N)
    s = jnp.sum(km * qv[...], axis=1, keepdims=True)
    m = jnp.max(s)
    e = jnp.exp(s - m)
    at = e / jnp.sum(e)
    a0 = at[0, 0]
    a1 = at[1, 0]
    rab = jnp.maximum(jnp.concatenate([ab0[...], ab1[...]], axis=1), 0.0)
    rob = jnp.maximum(jnp.concatenate([ob0[...], ob1[...]], axis=1), 0.0)
    out[...] = a0 * rab + a1 * rob


def _stage3(gab0, gab1, gob0, gob1, wk, bk, qv):
    rowh = pl.BlockSpec((BLK, 128), lambda i: (i, 0))
    full = lambda s: pl.BlockSpec(s, lambda i: (0, 0))
    ksum = pl.pallas_call(
        _s3a_body,
        grid=(GRID,),
        in_specs=[rowh, rowh, rowh, rowh, full((D, D)), full((1, D))],
        out_specs=pl.BlockSpec((2, D), lambda i: (0, 0)),
        out_shape=jax.ShapeDtypeStruct((2, D), jnp.float32),
    )(gab0, gab1, gob0, gob1, wk, bk)
    return pl.pallas_call(
        _s3b_body,
        grid=(GRID,),
        in_specs=[full((2, D)), full((1, D)), rowh, rowh, rowh, rowh],
        out_specs=pl.BlockSpec((BLK, D), lambda i: (i, 0)),
        out_shape=jax.ShapeDtypeStruct((N, D), jnp.float32),
    )(ksum, qv, gab0, gab1, gob0, gob1)


def _proj_mat(a):
    P = jnp.zeros((D, 16), jnp.float32)
    return P.at[jnp.arange(D), jnp.repeat(jnp.arange(H), DH)].set(a.reshape(-1))


def _pad_edges(ei):
    npad = EPAD - E
    src = (jnp.arange(npad, dtype=jnp.int32) * 7) % N
    dst = N + (jnp.arange(npad, dtype=jnp.int32) % 128)
    return jnp.concatenate([ei, jnp.stack([src, dst])], axis=1)


def kernel(x_abc_stock, x_other, edge_index_abc_to_abc, edge_index_other_to_abc,
           W_abc, b_abc, W_other, b_other, a_src_ab, a_dst_ab, a_src_ob,
           a_dst_ob, Wk, bk, q, emb_weight):
    ha0, ha1, ho0, ho1, tsa, tda, tso, tdo = _stage1(
        x_abc_stock, x_other, W_abc, W_other,
        b_abc.reshape(1, D), b_other.reshape(1, D),
        _proj_mat(a_src_ab), _proj_mat(a_dst_ab),
        _proj_mat(a_src_ob), _proj_mat(a_dst_ob))
    ei_ab = _pad_edges(edge_index_abc_to_abc).reshape(2, EPAD // ECH, ECH)
    ei_ob = _pad_edges(edge_index_other_to_abc).reshape(2, EPAD // ECH, ECH)
    gab0, gab1, gob0, gob1 = [g[:N] for g in _stage2(
        ei_ab, ei_ob, tsa, tda, tso, tdo, ha0, ha1, ho0, ho1)[:4]]
    out = _stage3(gab0, gab1, gob0, gob1, Wk, bk.reshape(1, D),
                  q.reshape(1, D))
    return out, emb_weight


# speedup vs baseline: 1.6363x; 1.0360x over previous
import functools

import jax
import jax.numpy as jnp
from jax import lax
from jax.experimental import pallas as pl
from jax.experimental.pallas import tpu as pltpu
from jax.experimental.pallas import tpu_sc as plsc

N = 10000
D = 256
H = 4
DH = 64
E = 160000
NC = 2
NS = 16
NPAD = 10240
EPAD = 163840
EPT = EPAD // NS
ECH = 128
NCHUNK = EPT // ECH
GRP = 8
NGRP = NCHUNK // GRP
RPT = NPAD // NS
ORT = N // NS
BLK = 1000
GRID = N // BLK


def _s1_body(xa, xo, wa, wo, ba, bo, psa, pda, pso, pdo,
             ha0, ha1, ho0, ho1, tsa, tda, tso, tdo):
    ha = jnp.dot(xa[...], wa[...], preferred_element_type=jnp.float32) + ba[...]
    ho = jnp.dot(xo[...], wo[...], preferred_element_type=jnp.float32) + bo[...]
    ha0[...] = ha[:, :128]
    ha1[...] = ha[:, 128:]
    ho0[...] = ho[:, :128]
    ho1[...] = ho[:, 128:]
    tsa[...] = jnp.dot(ha, psa[...], preferred_element_type=jnp.float32)
    tda[...] = jnp.dot(ha, pda[...], preferred_element_type=jnp.float32)
    tso[...] = jnp.dot(ho, pso[...], preferred_element_type=jnp.float32)
    tdo[...] = jnp.dot(ha, pdo[...], preferred_element_type=jnp.float32)


def _stage1(xa, xo, wa, wo, ba, bo, psa, pda, pso, pdo):
    row = pl.BlockSpec((BLK, D), lambda i: (i, 0))
    full = lambda s: pl.BlockSpec(s, lambda i: (0, 0))
    outs = [jax.ShapeDtypeStruct((N, 128), jnp.float32)] * 4 + \
           [jax.ShapeDtypeStruct((N, 16), jnp.float32)] * 4
    return pl.pallas_call(
        _s1_body,
        grid=(GRID,),
        in_specs=[row, row, full((D, D)), full((D, D)), full((1, D)),
                  full((1, D)), full((D, 16)), full((D, 16)), full((D, 16)),
                  full((D, 16))],
        out_specs=[pl.BlockSpec((BLK, 128), lambda i: (i, 0))] * 4 +
                  [pl.BlockSpec((BLK, 16), lambda i: (i, 0))] * 4,
        out_shape=outs,
    )(xa, xo, wa, wo, ba, bo, psa, pda, pso, pdo)


def _run_type(c, s, ei, tsrc, tdst, href, exref, recref, aggref,
              den, acc, sbuf, dbuf, arows, brows, ar2, br2, exch, rech,
              hrows, sem, sem2):
    ebase = s * EPT
    row0 = s * RPT

    @plsc.parallel_loop(0, 128, unroll=8)
    def zi(i):
        for u in range(8):
            hrows[i, pl.ds(u * 16, 16)] = jnp.zeros((16,), jnp.float32)
        arows[i, :] = jnp.zeros((16,), jnp.float32)
    for k in range(RPT // 128):
        pltpu.sync_copy(arows, den.at[pl.ds(row0 + k * 128, 128), :])
        pltpu.sync_copy(hrows, acc.at[pl.ds(row0 + k * 128, 128), :])
    plsc.subcore_barrier()

    crow0 = (s * EPT) // ECH
    abbufs = [(arows, brows, sem), (ar2, br2, sem2)]

    def p1(g, carry):
        gr = crow0 + g * GRP
        pltpu.sync_copy(ei.at[0, pl.ds(gr, GRP), :], sbuf)
        pltpu.sync_copy(ei.at[1, pl.ds(gr, GRP), :], dbuf)
        a0, b0, s0_ = abbufs[0]
        pltpu.async_copy(tsrc.at[sbuf.at[0]], a0, s0_)
        pltpu.async_copy(tdst.at[dbuf.at[0]], b0, s0_)
        for j in range(GRP):
            ar, br, sm = abbufs[j % 2]
            pltpu.make_async_copy(tsrc.at[sbuf.at[j]], ar, sm).wait()
            pltpu.make_async_copy(tdst.at[dbuf.at[j]], br, sm).wait()
            if j + 1 < GRP:
                an, bn, sn = abbufs[(j + 1) % 2]
                pltpu.async_copy(tsrc.at[sbuf.at[j + 1]], an, sn)
                pltpu.async_copy(tdst.at[dbuf.at[j + 1]], bn, sn)

            @plsc.parallel_loop(0, ECH, unroll=8)
            def cmp(i):
                a = ar[i, :] + br[i, :]
                a = jnp.maximum(a, 0.2 * a)
                exch[i, :] = jnp.exp(a)
            pltpu.sync_copy(exch, exref.at[pl.ds((gr + j) * ECH, ECH), :])
            pltpu.sync_copy(exch, den.at[dbuf.at[j]], add=True)
        return carry
    lax.fori_loop(0, NGRP, p1, 0)
    plsc.subcore_barrier()

    for k in range(RPT // 128):
        r0 = row0 + k * 128
        pltpu.sync_copy(den.at[pl.ds(r0, 128), :], arows)

        @plsc.parallel_loop(0, 128, unroll=8)
        def rcp(i):
            arows[i, :] = 1.0 / (arows[i, :] + 1e-16)
        pltpu.sync_copy(arows, recref.at[pl.ds(r0, 128), :])
    plsc.subcore_barrier()

    h0 = 2 * c
    i0 = jnp.broadcast_to(h0, (16, 1)).astype(jnp.int32)
    i1 = jnp.broadcast_to(h0 + 1, (16, 1)).astype(jnp.int32)
    _dnums = lax.GatherDimensionNumbers(
        offset_dims=(), collapsed_slice_dims=(0,), start_index_map=(0,))
    _splat = functools.partial(
        lax.gather, dimension_numbers=_dnums, slice_sizes=(1,),
        mode=lax.GatherScatterMode.PROMISE_IN_BOUNDS)

    def p3(g, carry):
        gr = crow0 + g * GRP
        pltpu.sync_copy(ei.at[0, pl.ds(gr, GRP), :], sbuf)
        pltpu.sync_copy(ei.at[1, pl.ds(gr, GRP), :], dbuf)

        def p3c(j, cc):
            eb = (gr + j) * ECH
            cp = pltpu.async_copy(href.at[sbuf.at[j]], hrows, sem)
            cr = pltpu.async_copy(recref.at[dbuf.at[j]], rech, sem2)
            pltpu.sync_copy(exref.at[pl.ds(eb, ECH), :], exch)
            cr.wait()
            cp.wait()

            @plsc.parallel_loop(0, ECH, unroll=4)
            def scale(k):
                row = exch[k, :] * rech[k, :]
                s0 = _splat(row, i0)
                s1 = _splat(row, i1)
                for u in range(4):
                    hrows[k, pl.ds(u * 16, 16)] = hrows[k, pl.ds(u * 16, 16)] * s0
                for u in range(4, 8):
                    hrows[k, pl.ds(u * 16, 16)] = hrows[k, pl.ds(u * 16, 16)] * s1
            pltpu.sync_copy(hrows, acc.at[dbuf.at[j]], add=True)
            return cc
        lax.fori_loop(0, GRP, p3c, 0)
        return carry
    lax.fori_loop(0, NGRP, p3, 0)
    plsc.subcore_barrier()

    for k in range(RPT // 128):
        r0 = row0 + k * 128
        pltpu.sync_copy(acc.at[pl.ds(r0, 128), :], hrows)
        pltpu.sync_copy(hrows, aggref.at[pl.ds(r0, 128), :])
    plsc.subcore_barrier()


def _sc_body(ei_ab, ei_ob, tsa, tda, tso, tdo, ha0, ha1, ho0, ho1,
             gab0, gab1, gob0, gob1, ex0, ex1, rec0, rec1,
             den, acc, sbuf, dbuf, arows, brows, ar2, br2, exch, rech,
             hrows, sem, sem2):
    c = lax.axis_index("c")
    s = lax.axis_index("s")

    common = (den, acc, sbuf, dbuf, arows, brows, ar2, br2, exch, rech,
              hrows, sem, sem2)

    @pl.when(c == 0)
    def _():
        _run_type(c, s, ei_ab, tsa, tda, ha0, ex0, rec0, gab0, *common)
        _run_type(c, s, ei_ob, tso, tdo, ho0, ex0, rec0, gob0, *common)

    @pl.when(c == 1)
    def _():
        _run_type(c, s, ei_ab, tsa, tda, ha1, ex1, rec1, gab1, *common)
        _run_type(c, s, ei_ob, tso, tdo, ho1, ex1, rec1, gob1, *common)


def _stage2(ei_ab, ei_ob, tsa, tda, tso, tdo, ha0, ha1, ho0, ho1):
    mesh = plsc.VectorSubcoreMesh(core_axis_name="c", subcore_axis_name="s",
                                  num_cores=NC, num_subcores=NS)
    f32 = jnp.float32
    out_type = (
        jax.ShapeDtypeStruct((NPAD, 128), f32),
        jax.ShapeDtypeStruct((NPAD, 128), f32),
        jax.ShapeDtypeStruct((NPAD, 128), f32),
        jax.ShapeDtypeStruct((NPAD, 128), f32),
        jax.ShapeDtypeStruct((EPAD, 16), f32),
        jax.ShapeDtypeStruct((EPAD, 16), f32),
        jax.ShapeDtypeStruct((NPAD, 16), f32),
        jax.ShapeDtypeStruct((NPAD, 16), f32),
    )
    scratch = [
        pltpu.VMEM_SHARED((NPAD, 16), f32),
        pltpu.VMEM_SHARED((NPAD, 128), f32),
        pltpu.VMEM((GRP, ECH), jnp.int32),
        pltpu.VMEM((GRP, ECH), jnp.int32),
        pltpu.VMEM((ECH, 16), f32),
        pltpu.VMEM((ECH, 16), f32),
        pltpu.VMEM((ECH, 16), f32),
        pltpu.VMEM((ECH, 16), f32),
        pltpu.VMEM((ECH, 16), f32),
        pltpu.VMEM((ECH, 16), f32),
        pltpu.VMEM((ECH, 128), f32),
        pltpu.SemaphoreType.DMA,
        pltpu.SemaphoreType.DMA,
    ]
    fn = pl.kernel(_sc_body, out_type=out_type, mesh=mesh,
                   scratch_types=scratch,
                   compiler_params=pltpu.CompilerParams(
                       use_tc_tiling_on_sc=False))
    return fn(ei_ab, ei_ob, tsa, tda, tso, tdo, ha0, ha1, ho0, ho1)


def _s3a_body(ab0, ab1, ob0, ob1, wk, bk, ksum):
    i = pl.program_id(0)
    rab = jnp.maximum(jnp.concatenate([ab0[...], ab1[...]], axis=1), 0.0)
    rob = jnp.maximum(jnp.concatenate([ob0[...], ob1[...]], axis=1), 0.0)
    tab = jnp.tanh(jnp.dot(rab, wk[...], preferred_element_type=jnp.float32) + bk[...])
    tob = jnp.tanh(jnp.dot(rob, wk[...], preferred_element_type=jnp.float32) + bk[...])
    blk = jnp.concatenate([jnp.sum(tab, axis=0, keepdims=True),
                           jnp.sum(tob, axis=0, keepdims=True)], axis=0)

    @pl.when(i == 0)
    def _():
        ksum[...] = blk

    @pl.when(i > 0)
    def _():
        ksum[...] = ksum[...] + blk


def _s3b_body(ksum, qv, ab0, ab1, ob0, ob1, out):
    km = ksum[...] * (1.0 / N)
    s = jnp.sum(km * qv[...], axis=1, keepdims=True)
    m = jnp.max(s)
    e = jnp.exp(s - m)
    at = e / jnp.sum(e)
    a0 = at[0, 0]
    a1 = at[1, 0]
    rab = jnp.maximum(jnp.concatenate([ab0[...], ab1[...]], axis=1), 0.0)
    rob = jnp.maximum(jnp.concatenate([ob0[...], ob1[...]], axis=1), 0.0)
    out[...] = a0 * rab + a1 * rob


def _stage3(gab0, gab1, gob0, gob1, wk, bk, qv):
    rowh = pl.BlockSpec((BLK, 128), lambda i: (i, 0))
    full = lambda s: pl.BlockSpec(s, lambda i: (0, 0))
    ksum = pl.pallas_call(
        _s3a_body,
        grid=(GRID,),
        in_specs=[rowh, rowh, rowh, rowh, full((D, D)), full((1, D))],
        out_specs=pl.BlockSpec((2, D), lambda i: (0, 0)),
        out_shape=jax.ShapeDtypeStruct((2, D), jnp.float32),
    )(gab0, gab1, gob0, gob1, wk, bk)
    return pl.pallas_call(
        _s3b_body,
        grid=(GRID,),
        in_specs=[full((2, D)), full((1, D)), rowh, rowh, rowh, rowh],
        out_specs=pl.BlockSpec((BLK, D), lambda i: (i, 0)),
        out_shape=jax.ShapeDtypeStruct((N, D), jnp.float32),
    )(ksum, qv, gab0, gab1, gob0, gob1)


def _proj_mat(a):
    P = jnp.zeros((D, 16), jnp.float32)
    return P.at[jnp.arange(D), jnp.repeat(jnp.arange(H), DH)].set(a.reshape(-1))


def _pad_edges(ei):
    npad = EPAD - E
    src = (jnp.arange(npad, dtype=jnp.int32) * 7) % N
    dst = N + (jnp.arange(npad, dtype=jnp.int32) % 128)
    return jnp.concatenate([ei, jnp.stack([src, dst])], axis=1)


def kernel(x_abc_stock, x_other, edge_index_abc_to_abc, edge_index_other_to_abc,
           W_abc, b_abc, W_other, b_other, a_src_ab, a_dst_ab, a_src_ob,
           a_dst_ob, Wk, bk, q, emb_weight):
    ha0, ha1, ho0, ho1, tsa, tda, tso, tdo = _stage1(
        x_abc_stock, x_other, W_abc, W_other,
        b_abc.reshape(1, D), b_other.reshape(1, D),
        _proj_mat(a_src_ab), _proj_mat(a_dst_ab),
        _proj_mat(a_src_ob), _proj_mat(a_dst_ob))
    ei_ab = _pad_edges(edge_index_abc_to_abc).reshape(2, EPAD // ECH, ECH)
    ei_ob = _pad_edges(edge_index_other_to_abc).reshape(2, EPAD // ECH, ECH)
    gab0, gab1, gob0, gob1 = [g[:N] for g in _stage2(
        ei_ab, ei_ob, tsa, tda, tso, tdo, ha0, ha1, ho0, ho1)[:4]]
    out = _stage3(gab0, gab1, gob0, gob1, Wk, bk.reshape(1, D),
                  q.reshape(1, D))
    return out, emb_weight
